# Initial kernel scaffold; baseline (speedup 1.0000x reference)
import jax
import jax.numpy as jnp
from jax.experimental import pallas as pl


def kernel(x, bias_table):
    raise NotImplementedError("write your pallas kernel here")


# speedup vs baseline: 40.1975x; 40.1975x over previous
import functools

import jax
import jax.numpy as jnp
from jax import lax
from jax.experimental import pallas as pl
from jax.experimental.pallas import tpu as pltpu
from jax.experimental.pallas import tpu_sc as plsc

L = 2048
H = 16
NSHIFT = 16
TW = 4096
NC = 2
NS = 16
CHUNK = 8


def _sc_body(p_hbm, out_hbm, p_v, sem):
    cid = lax.axis_index("c")
    sid = lax.axis_index("s")
    wid = sid * NC + cid
    h = wid // 2
    half = wid % 2
    base_i = half * (L // 2)

    pltpu.sync_copy(p_hbm.at[h], p_v)

    n_chunks = (L // 2) // CHUNK

    def fire(chunk_idx):
        for b in range(CHUNK):
            i = base_i + chunk_idx * CHUNK + b
            o = (L - 1) - i
            r = lax.rem(o, NSHIFT)
            a = pl.multiple_of(o - r, NSHIFT)
            pltpu.async_copy(
                p_v.at[r, pl.ds(a, L)],
                out_hbm.at[0, h, i],
                sem,
            )

    def drain_one_chunk():
        pltpu.make_async_copy(
            p_v.at[pl.ds(0, CHUNK), pl.ds(0, L)],
            out_hbm.at[0, 0, pl.ds(0, CHUNK)],
            sem,
        ).wait()

    def loop_body(g, carry):
        fire(g)

        @pl.when(g > 0)
        def _():
            drain_one_chunk()

        return carry

    lax.fori_loop(0, n_chunks, loop_body, 0)
    drain_one_chunk()


@functools.partial(jax.jit, static_argnames=())
def _run_sc(p):
    mesh = plsc.VectorSubcoreMesh(
        core_axis_name="c", subcore_axis_name="s", num_cores=NC, num_subcores=NS
    )
    return pl.kernel(
        _sc_body,
        out_type=jax.ShapeDtypeStruct((1, H, L, L), jnp.float32),
        mesh=mesh,
        scratch_types=[
            pltpu.VMEM((NSHIFT, TW), jnp.float32),
            pltpu.SemaphoreType.DMA,
        ],
        compiler_params=pltpu.CompilerParams(use_tc_tiling_on_sc=False),
    )(p)


def kernel(x, bias_table):
    del x
    bp = jnp.pad(bias_table, ((0, NSHIFT + TW - (2 * L - 1)), (0, 0)))
    p = jnp.stack([lax.slice(bp, (r, 0), (r + TW, H)) for r in range(NSHIFT)])
    p = jnp.transpose(p, (2, 0, 1))
    return _run_sc(p)


# speedup vs baseline: 40.5265x; 1.0082x over previous
import functools

import jax
import jax.numpy as jnp
from jax import lax
from jax.experimental import pallas as pl
from jax.experimental.pallas import tpu as pltpu
from jax.experimental.pallas import tpu_sc as plsc

L = 2048
H = 16
NSHIFT = 16
TW = 4096
NC = 2
NS = 16
CHUNK = 8


def _sc_body(p_hbm, out_hbm, p_v, sem):
    cid = lax.axis_index("c")
    sid = lax.axis_index("s")
    wid = sid * NC + cid
    h = wid // 2
    half = wid % 2
    base_i = half * (L // 2)

    pltpu.sync_copy(p_hbm.at[h], p_v)

    n_groups = (L // 2) // NSHIFT

    def fire(g):
        i0 = base_i + g * NSHIFT
        a = pl.multiple_of((L - NSHIFT) - i0, NSHIFT)
        pltpu.async_copy(
            p_v.at[:, pl.ds(a, L)],
            out_hbm.at[0, h, pl.ds(i0, NSHIFT)],
            sem,
        )

    def drain_one_group():
        pltpu.make_async_copy(
            p_v.at[:, pl.ds(0, L)],
            out_hbm.at[0, 0, pl.ds(0, NSHIFT)],
            sem,
        ).wait()

    def loop_body(g, carry):
        fire(g)

        @pl.when(g > 0)
        def _():
            drain_one_group()

        return carry

    lax.fori_loop(0, n_groups, loop_body, 0)
    drain_one_group()


@functools.partial(jax.jit, static_argnames=())
def _run_sc(p):
    mesh = plsc.VectorSubcoreMesh(
        core_axis_name="c", subcore_axis_name="s", num_cores=NC, num_subcores=NS
    )
    return pl.kernel(
        _sc_body,
        out_type=jax.ShapeDtypeStruct((1, H, L, L), jnp.float32),
        mesh=mesh,
        scratch_types=[
            pltpu.VMEM((NSHIFT, TW), jnp.float32),
            pltpu.SemaphoreType.DMA,
        ],
        compiler_params=pltpu.CompilerParams(use_tc_tiling_on_sc=False),
    )(p)


def kernel(x, bias_table):
    del x
    bp = jnp.pad(bias_table, ((0, NSHIFT + TW - (2 * L - 1)), (0, 0)))
    p = jnp.stack(
        [lax.slice(bp, (NSHIFT - 1 - q, 0), (NSHIFT - 1 - q + TW, H))
         for q in range(NSHIFT)])
    p = jnp.transpose(p, (2, 0, 1))
    return _run_sc(p)
